# baseline (speedup 1.0000x reference)
import functools

import numpy as np
import jax
import jax.numpy as jnp
from jax import lax
from jax.experimental import pallas as pl
from jax.experimental.pallas import tpu as pltpu
from jax.experimental.pallas import tpu_sc as plsc

ENT_TOT = 1000000
REL_TOT = 1000
REL_PAD = 1024
DIM = 64
PK = 2 * DIM
BATCH = 16384

NUM_CORES = 2
NUM_SUBCORES = 16
NUM_WORKERS = NUM_CORES * NUM_SUBCORES
B_PER_W = BATCH // NUM_WORKERS
CHUNK = 128
N_CHUNKS = B_PER_W // CHUNK
GROUPS = CHUNK // 16

PACK_BLK = 5000

_TAKE_DNUMS = lax.GatherDimensionNumbers(
    offset_dims=(), collapsed_slice_dims=(0,), start_index_map=(0,))


def _lane_permute(v, perm):
    return lax.gather(v, perm[:, None], dimension_numbers=_TAKE_DNUMS,
                      slice_sizes=(1,),
                      mode=lax.GatherScatterMode.PROMISE_IN_BOUNDS)


def _lane_sum(v, perms):
    for perm in perms:
        v = v + _lane_permute(v, perm)
    return v


def _rsqrt_nr(x):
    xi = lax.bitcast_convert_type(x, jnp.int32)
    yi = 0x5F3759DF - lax.shift_right_arithmetic(xi, 1)
    y = lax.bitcast_convert_type(yi, jnp.float32)
    for _ in range(2):
        y = y * (1.5 - 0.5 * x * y * y)
    return y


def _pack_body(a_ref, b_ref, o_ref):
    o_ref[...] = jnp.concatenate([a_ref[...], b_ref[...]], axis=1)


def _pack_halves(table, rows, blk):
    half_blocks = (rows // 2) // blk
    return pl.pallas_call(
        _pack_body,
        grid=(half_blocks,),
        in_specs=[
            pl.BlockSpec((blk, DIM), lambda i: (i, 0)),
            pl.BlockSpec((blk, DIM), lambda i, hb=half_blocks: (i + hb, 0)),
        ],
        out_specs=pl.BlockSpec((blk, PK), lambda i: (i, 0)),
        out_shape=jax.ShapeDtypeStruct((rows // 2, PK), jnp.float32),
    )(table, table)


def _transe_sc(batch_h, batch_t, batch_r, ent_p, rel_p):
    mesh = plsc.VectorSubcoreMesh(core_axis_name="c", subcore_axis_name="s")

    @functools.partial(
        pl.kernel,
        mesh=mesh,
        out_type=jax.ShapeDtypeStruct((BATCH,), jnp.float32),
        scratch_types=[
            pltpu.VMEM((N_CHUNKS, CHUNK), jnp.int32),
            pltpu.VMEM((N_CHUNKS, CHUNK), jnp.int32),
            pltpu.VMEM((N_CHUNKS, CHUNK), jnp.int32),
            pltpu.VMEM((N_CHUNKS, CHUNK), jnp.int32),
            pltpu.VMEM((N_CHUNKS, CHUNK), jnp.int32),
            pltpu.VMEM((N_CHUNKS, CHUNK), jnp.int32),
            pltpu.VMEM((2, CHUNK, PK), jnp.float32),
            pltpu.VMEM((2, CHUNK, PK), jnp.float32),
            pltpu.VMEM((2, CHUNK, PK), jnp.float32),
            pltpu.VMEM((B_PER_W,), jnp.float32),
            pltpu.SemaphoreType.DMA,
            pltpu.SemaphoreType.DMA,
        ],
    )
    def k(bh_hbm, bt_hbm, br_hbm, ent_hbm, rel_hbm, out_hbm,
          ri_h, ri_t, ri_r, pi_h, pi_t, pi_r,
          h_buf, t_buf, r_buf, out_v, sem0, sem1):
        wid = lax.axis_index("s") * NUM_CORES + lax.axis_index("c")
        base = wid * B_PER_W
        sems = (sem0, sem1)

        for c in range(N_CHUNKS):
            off = base + c * CHUNK
            pltpu.sync_copy(bh_hbm.at[pl.ds(off, CHUNK)], ri_h.at[c])
            pltpu.sync_copy(bt_hbm.at[pl.ds(off, CHUNK)], ri_t.at[c])
            pltpu.sync_copy(br_hbm.at[pl.ds(off, CHUNK)], ri_r.at[c])
        for raw, packed, half in ((ri_h, pi_h, ENT_TOT // 2),
                                  (ri_t, pi_t, ENT_TOT // 2),
                                  (ri_r, pi_r, REL_PAD // 2)):
            hv = jnp.int32(half)
            for c in range(N_CHUNKS):
                for v in range(CHUNK // 16):
                    sl = pl.ds(v * 16, 16)
                    x = raw[c, sl]
                    packed[c, sl] = jnp.where(x >= hv, x - hv, x)

        iota16 = lax.iota(jnp.int32, 16)
        perms = [lax.bitwise_xor(iota16, jnp.int32(kk)) for kk in (1, 2, 4, 8)]

        def fire(c, b):
            return [
                pltpu.async_copy(ent_hbm.at[pi_h.at[c]], h_buf.at[b], sems[b]),
                pltpu.async_copy(ent_hbm.at[pi_t.at[c]], t_buf.at[b], sems[b]),
                pltpu.async_copy(rel_hbm.at[pi_r.at[c]], r_buf.at[b], sems[b]),
            ]

        def compute(c, b):
            def group_body(g, _):
                sl = pl.ds(c * CHUNK + g * 16, 16)
                del sl
                hvec = ri_h[c, pl.ds(g * 16, 16)]
                tvec = ri_t[c, pl.ds(g * 16, 16)]
                rvec = ri_r[c, pl.ds(g * 16, 16)]
                acc = jnp.zeros((16,), jnp.float32)
                for j in range(16):
                    def _off(x, half):
                        ge = (x >= jnp.int32(half)).astype(jnp.int32)
                        return pl.multiple_of(ge * DIM, DIM)

                    oh = _off(hvec[j], ENT_TOT // 2)
                    ot = _off(tvec[j], ENT_TOT // 2)
                    orr = _off(rvec[j], REL_PAD // 2)
                    row = g * 16 + j
                    hv = [h_buf[b, row, pl.ds(oh + 16 * q, 16)]
                          for q in range(4)]
                    tv = [t_buf[b, row, pl.ds(ot + 16 * q, 16)]
                          for q in range(4)]
                    rv = [r_buf[b, row, pl.ds(orr + 16 * q, 16)]
                          for q in range(4)]

                    def inv_norm(vs):
                        ssq = ((vs[0] * vs[0] + vs[1] * vs[1])
                               + (vs[2] * vs[2] + vs[3] * vs[3]))
                        return _rsqrt_nr(
                            jnp.maximum(_lane_sum(ssq, perms), 1e-24))

                    ih = inv_norm(hv)
                    it = inv_norm(tv)
                    ir = inv_norm(rv)

                    s = jnp.zeros((16,), jnp.float32)
                    for q in range(4):
                        s = s + jnp.abs(hv[q] * ih + rv[q] * ir - tv[q] * it)
                    score = _lane_sum(s, perms)
                    acc = jnp.where(iota16 == j, score, acc)
                out_v[pl.ds(c * CHUNK + g * 16, 16)] = acc
                return 0

            lax.fori_loop(0, GROUPS, group_body, 0)

        cps0 = fire(0, 0)
        cps1 = fire(1, 1)
        for cp in cps0:
            cp.wait()
        compute(0, 0)
        cps2 = fire(2, 0)
        for cp in cps1:
            cp.wait()
        compute(1, 1)
        cps3 = fire(3, 1)
        for cp in cps2:
            cp.wait()
        compute(2, 0)
        for cp in cps3:
            cp.wait()
        compute(3, 1)

        pltpu.sync_copy(out_v, out_hbm.at[pl.ds(base, B_PER_W)])

    return k(batch_h, batch_t, batch_r, ent_p, rel_p)


def kernel(batch_h, batch_t, batch_r, ent_emb, rel_emb):
    ent_p = _pack_halves(ent_emb, ENT_TOT, PACK_BLK)
    rel_pad = jnp.pad(rel_emb, ((0, REL_PAD - REL_TOT), (0, 0)))
    rel_p = _pack_halves(rel_pad, REL_PAD, REL_PAD // 2)
    return _transe_sc(batch_h, batch_t, batch_r, ent_p, rel_p)
